# baseline (speedup 1.0000x reference)
import functools

import jax
import jax.numpy as jnp
from jax import lax
from jax.experimental import pallas as pl
from jax.experimental.pallas import tpu as pltpu
from jax.experimental.pallas import tpu_sc as plsc

NUM_CORES = 2
NUM_SUBCORES = 16
NW = NUM_CORES * NUM_SUBCORES
LANES = 16

B = 16384
D = 64
BPW = B // NW
CHUNK = 128
NCHUNK = BPW // CHUNK
NGROUP = BPW // LANES


def _sc_call(users2, items2, user_table, item_table):
    mesh = plsc.VectorSubcoreMesh(core_axis_name="c", subcore_axis_name="s")

    @functools.partial(
        pl.kernel,
        mesh=mesh,
        out_type=jax.ShapeDtypeStruct((B,), jnp.float32),
        compiler_params=pltpu.CompilerParams(
            needs_layout_passes=False, use_tc_tiling_on_sc=False),
        scratch_types=[
            pltpu.VMEM((NCHUNK, CHUNK), jnp.int32),
            pltpu.VMEM((NCHUNK, CHUNK), jnp.int32),
            pltpu.VMEM((BPW, D), jnp.float32),
            pltpu.VMEM((BPW, D), jnp.float32),
            pltpu.VMEM((BPW,), jnp.float32),
            pltpu.SemaphoreType.DMA,
        ],
    )
    def k(u_hbm, i_hbm, ut_hbm, it_hbm, out_hbm,
          uidx_v, iidx_v, urows_v, irows_v, sc_v, sem):
        wid = lax.axis_index("s") * NUM_CORES + lax.axis_index("c")
        base = wid * BPW
        pltpu.sync_copy(u_hbm.at[wid], uidx_v)
        pltpu.sync_copy(i_hbm.at[wid], iidx_v)
        copies = []
        for j in range(NCHUNK):
            copies.append(pltpu.async_copy(
                ut_hbm.at[uidx_v.at[j]],
                urows_v.at[pl.ds(j * CHUNK, CHUNK)], sem))
            copies.append(pltpu.async_copy(
                it_hbm.at[iidx_v.at[j]],
                irows_v.at[pl.ds(j * CHUNK, CHUNK)], sem))
        for c in copies:
            c.wait()

        lanes = lax.iota(jnp.int32, LANES)

        def group_body(g, carry):
            dots = jnp.zeros((LANES,), jnp.float32)
            for k in range(LANES):
                r = g * LANES + k
                acc = jnp.zeros((LANES,), jnp.float32)
                for c in range(D // LANES):
                    u = urows_v[r, pl.ds(c * LANES, LANES)]
                    v = irows_v[r, pl.ds(c * LANES, LANES)]
                    acc = acc + u * v
                dots = jnp.where(lanes == k, jnp.sum(acc), dots)
            sc_v[pl.ds(g * LANES, LANES)] = 1.0 / (1.0 + jnp.exp(-dots))
            return carry

        lax.fori_loop(0, NGROUP, group_body, 0)
        pltpu.sync_copy(sc_v, out_hbm.at[pl.ds(base, BPW)])

    return k(users2, items2, user_table, item_table)


def kernel(users, items, user_table, item_table):
    users2 = users.reshape(NW, NCHUNK, CHUNK)
    items2 = items.reshape(NW, NCHUNK, CHUNK)
    return _sc_call(users2, items2, user_table, item_table)


# speedup vs baseline: 2.3513x; 2.3513x over previous
import functools

import jax
import jax.numpy as jnp
from jax import lax
from jax.experimental import pallas as pl
from jax.experimental.pallas import tpu as pltpu
from jax.experimental.pallas import tpu_sc as plsc

NUM_CORES = 2
NUM_SUBCORES = 16
NW = NUM_CORES * NUM_SUBCORES
LANES = 16

B = 16384
D = 64
BPW = B // NW
NGROUP = BPW // LANES
NSLAB = 4


def _sc_call(users2, items2, ut_t, it_t):
    mesh = plsc.VectorSubcoreMesh(core_axis_name="c", subcore_axis_name="s")

    @functools.partial(
        pl.kernel,
        mesh=mesh,
        out_type=jax.ShapeDtypeStruct((NW, BPW), jnp.float32),
        compiler_params=pltpu.CompilerParams(
            needs_layout_passes=False, use_tc_tiling_on_sc=True),
        scratch_types=[
            pltpu.VMEM((BPW,), jnp.int32),
            pltpu.VMEM((BPW,), jnp.int32),
            pltpu.VMEM((NSLAB, D, 128), jnp.float32),
            pltpu.VMEM((NSLAB, D, 128), jnp.float32),
            pltpu.VMEM((BPW,), jnp.float32),
            pltpu.SemaphoreType.DMA,
        ],
    )
    def k(u_hbm, i_hbm, ut_hbm, it_hbm, out_hbm,
          uidx_v, iidx_v, uslab_v, islab_v, sc_v, sem):
        wid = lax.axis_index("s") * NUM_CORES + lax.axis_index("c")
        pltpu.sync_copy(u_hbm.at[wid], uidx_v)
        pltpu.sync_copy(i_hbm.at[wid], iidx_v)

        lanes = lax.iota(jnp.int32, LANES)

        def round_body(g, carry):
            vu = uidx_v[pl.ds(g * LANES, LANES)]
            vi = iidx_v[pl.ds(g * LANES, LANES)]
            dots = jnp.zeros((LANES,), jnp.float32)
            for sub in range(LANES // NSLAB):
                copies = []
                for e in range(NSLAB):
                    kk = sub * NSLAB + e
                    ublk = pl.multiple_of((vu[kk] >> 7) * 128, 128)
                    iblk = pl.multiple_of((vi[kk] >> 7) * 128, 128)
                    copies.append(pltpu.async_copy(
                        ut_hbm.at[:, pl.ds(ublk, 128)], uslab_v.at[e], sem))
                    copies.append(pltpu.async_copy(
                        it_hbm.at[:, pl.ds(iblk, 128)], islab_v.at[e], sem))
                for c in copies:
                    c.wait()
                for e in range(NSLAB):
                    kk = sub * NSLAB + e
                    ucol = jnp.full((LANES,), vu[kk] & 127, jnp.int32)
                    icol = jnp.full((LANES,), vi[kk] & 127, jnp.int32)
                    slab = jnp.full((LANES,), e, jnp.int32)
                    acc = jnp.zeros((LANES,), jnp.float32)
                    for c in range(D // LANES):
                        dvec = lanes + c * LANES
                        gu = plsc.load_gather(uslab_v, [slab, dvec, ucol])
                        gi = plsc.load_gather(islab_v, [slab, dvec, icol])
                        acc = acc + gu * gi
                    dots = jnp.where(lanes == kk, jnp.sum(acc), dots)
            sc_v[pl.ds(g * LANES, LANES)] = 1.0 / (1.0 + jnp.exp(-dots))
            return carry

        lax.fori_loop(0, NGROUP, round_body, 0)
        pltpu.sync_copy(sc_v, out_hbm.at[wid])

    return k(users2, items2, ut_t, it_t)


def kernel(users, items, user_table, item_table):
    users2 = users.reshape(NW, BPW)
    items2 = items.reshape(NW, BPW)
    out = _sc_call(users2, items2, user_table.T, item_table.T)
    return out.reshape(B)


# speedup vs baseline: 2.7724x; 1.1791x over previous
import functools

import jax
import jax.numpy as jnp
from jax import lax
from jax.experimental import pallas as pl
from jax.experimental.pallas import tpu as pltpu
from jax.experimental.pallas import tpu_sc as plsc

NUM_CORES = 2
NUM_SUBCORES = 16
NW = NUM_CORES * NUM_SUBCORES
LANES = 16

B = 16384
D = 64
BPW = B // NW
NGROUP = BPW // LANES
NSLAB = 4


def _sc_call(users2, items2, perm2, ut_t, it_t):
    mesh = plsc.VectorSubcoreMesh(core_axis_name="c", subcore_axis_name="s")

    @functools.partial(
        pl.kernel,
        mesh=mesh,
        out_type=jax.ShapeDtypeStruct((B,), jnp.float32),
        compiler_params=pltpu.CompilerParams(
            needs_layout_passes=False, use_tc_tiling_on_sc=True),
        scratch_types=[
            pltpu.VMEM((BPW,), jnp.int32),
            pltpu.VMEM((BPW,), jnp.int32),
            pltpu.VMEM((BPW,), jnp.int32),
            pltpu.VMEM((2 * NSLAB, D, 128), jnp.float32),
            pltpu.VMEM((NSLAB, D, 128), jnp.float32),
            pltpu.VMEM((BPW,), jnp.float32),
            pltpu.SemaphoreType.DMA,
            pltpu.SemaphoreType.DMA,
        ],
    )
    def k(u_hbm, i_hbm, p_hbm, ut_hbm, it_hbm, out_hbm,
          uidx_v, iidx_v, pidx_v, uslab_v, islab_v, sc_v, sem, osem):
        wid = lax.axis_index("s") * NUM_CORES + lax.axis_index("c")
        pltpu.sync_copy(u_hbm.at[wid], uidx_v)
        pltpu.sync_copy(i_hbm.at[wid], iidx_v)
        pltpu.sync_copy(p_hbm.at[wid], pidx_v)

        lanes = lax.iota(jnp.int32, LANES)

        def round_body(g, carry):
            prev_blk, prev_slot, fcnt = carry
            vu = uidx_v[pl.ds(g * LANES, LANES)]
            vi = iidx_v[pl.ds(g * LANES, LANES)]
            dots = jnp.zeros((LANES,), jnp.float32)
            for sub in range(LANES // NSLAB):
                blks, slots, changed = [], [], []
                for e in range(NSLAB):
                    kk = sub * NSLAB + e
                    blk = vu[kk] >> 7
                    ch = blk != prev_blk
                    slot = jnp.where(ch, fcnt & 7, prev_slot)
                    fcnt = fcnt + ch.astype(jnp.int32)
                    blks.append(blk)
                    slots.append(slot)
                    changed.append(ch)
                    prev_blk, prev_slot = blk, slot
                for e in range(NSLAB):
                    kk = sub * NSLAB + e

                    @pl.when(changed[e])
                    def _(e=e):
                        ublk = pl.multiple_of(blks[e] * 128, 128)
                        pltpu.async_copy(
                            ut_hbm.at[:, pl.ds(ublk, 128)],
                            uslab_v.at[slots[e]], sem)

                    iblk = pl.multiple_of((vi[kk] >> 7) * 128, 128)
                    pltpu.async_copy(
                        it_hbm.at[:, pl.ds(iblk, 128)], islab_v.at[e], sem)
                for e in range(NSLAB):
                    kk = sub * NSLAB + e

                    @pl.when(changed[e])
                    def _(e=e):
                        ublk = pl.multiple_of(blks[e] * 128, 128)
                        pltpu.make_async_copy(
                            ut_hbm.at[:, pl.ds(ublk, 128)],
                            uslab_v.at[slots[e]], sem).wait()

                    iblk = pl.multiple_of((vi[kk] >> 7) * 128, 128)
                    pltpu.make_async_copy(
                        it_hbm.at[:, pl.ds(iblk, 128)],
                        islab_v.at[e], sem).wait()
                for e in range(NSLAB):
                    kk = sub * NSLAB + e
                    ucol = jnp.full((LANES,), vu[kk] & 127, jnp.int32)
                    icol = jnp.full((LANES,), vi[kk] & 127, jnp.int32)
                    uslb = jnp.full((LANES,), slots[e], jnp.int32)
                    islb = jnp.full((LANES,), e, jnp.int32)
                    acc = jnp.zeros((LANES,), jnp.float32)
                    for c in range(D // LANES):
                        dvec = lanes + c * LANES
                        gu = plsc.load_gather(uslab_v, [uslb, dvec, ucol])
                        gi = plsc.load_gather(islab_v, [islb, dvec, icol])
                        acc = acc + gu * gi
                    dots = jnp.where(lanes == kk, jnp.sum(acc), dots)
            sc_v[pl.ds(g * LANES, LANES)] = 1.0 / (1.0 + jnp.exp(-dots))
            return prev_blk, prev_slot, fcnt

        lax.fori_loop(0, NGROUP, round_body,
                      (jnp.int32(-1), jnp.int32(0), jnp.int32(0)))
        pltpu.async_copy(sc_v, out_hbm.at[pidx_v], osem).wait()

    return k(users2, items2, perm2, ut_t, it_t)


def kernel(users, items, user_table, item_table):
    perm = lax.iota(jnp.int32, B)
    users_s, items_s, perm_s = lax.sort(
        (users, items, perm), dimension=0, num_keys=1)
    out = _sc_call(
        users_s.reshape(NW, BPW),
        items_s.reshape(NW, BPW),
        perm_s.reshape(NW, BPW),
        user_table.T, item_table.T)
    return out
